# baseline (speedup 1.0000x reference)
import jax
import jax.numpy as jnp
from jax import lax
from jax.experimental import pallas as pl
from jax.experimental.pallas import tpu as pltpu
from jax.experimental.pallas import tpu_sc as plsc

_NC, _NS = 2, 16
_NW = _NC * _NS

_CHUNK_A = 104
_RG = 8


def _make_body(B, L, D, V, M):
    ROWS = B // _NW
    NFULL = L // 16
    TAIL0 = L - 16
    TAILLO = TAIL0 - 16 * (NFULL - 1)
    CHB = L - _CHUNK_A

    PAD = ((L + 15) // 16) * 16

    def body(ids_hbm, ws_hbm, tok_hbm, sm_hbm, out_hbm,
             ids8_v, ws8_v, idx2_v, p2_v, tok2_v, pos_v,
             word_v, wse_v, gam_v, bet_v, out_v, sem):
        wid = lax.axis_index("s") * _NC + lax.axis_index("c")
        lane = lax.iota(jnp.int32, 16)

        o_pos, o_word, o_wse, o_gam, o_bet = (
            0, L * D, L * D + M * D, L * D + M * D + 2 * D,
            L * D + M * D + 2 * D + D)
        pltpu.sync_copy(sm_hbm.at[pl.ds(o_pos, L * D)], pos_v)
        pltpu.sync_copy(sm_hbm.at[pl.ds(o_word, M * D)], word_v)
        pltpu.sync_copy(sm_hbm.at[pl.ds(o_wse, 2 * D)], wse_v)
        pltpu.sync_copy(sm_hbm.at[pl.ds(o_gam, D)], gam_v)
        pltpu.sync_copy(sm_hbm.at[pl.ds(o_bet, D)], bet_v)

        ws0 = [wse_v[pl.ds(16 * j, 16)] for j in range(4)]
        wsd = [wse_v[pl.ds(D + 16 * j, 16)] - ws0[j] for j in range(4)]
        gam = [gam_v[pl.ds(16 * j, 16)] for j in range(4)]
        bet = [bet_v[pl.ds(16 * j, 16)] for j in range(4)]

        def stage_group(b8):
            b8 = pl.multiple_of(b8, _RG)
            pltpu.sync_copy(ids_hbm.at[pl.ds(b8, _RG)], ids8_v)
            pltpu.sync_copy(ws_hbm.at[pl.ds(b8, _RG)], ws8_v)

        def pre_row(rnxt):
            i = rnxt & (_RG - 1)
            pbase = pl.multiple_of((rnxt & 1) * PAD, 16)

            def do_chunk(dst16, v, idv, runbc):
                cum = plsc.cumsum(v) + runbc
                p2_v[dst16] = lax.shift_left(cum, 8) | lax.shift_left(v, 6)
                idx2_v[dst16] = idv

            run = jnp.int32(0)
            s_tail = jnp.int32(0)
            for k in range(NFULL):
                v = ws8_v[i, pl.ds(16 * k, 16)]
                idv = ids8_v[i, pl.ds(16 * k, 16)]
                do_chunk(pl.ds(pbase + 16 * k, 16), v, idv, lax.broadcast(run, (16,)))
                if k == NFULL - 1:
                    s_tail = run + jnp.sum(jnp.where(lane < TAILLO, v, 0))
                run = run + jnp.sum(v)
            if L > 16 * NFULL:
                vB = ws8_v[i, pl.ds(TAIL0, 16)]
                idvB = ids8_v[i, pl.ds(TAIL0, 16)]
                do_chunk(pl.ds(pbase + TAIL0, 16), vB, idvB,
                         lax.broadcast(s_tail, (16,)))

            pltpu.async_copy(
                tok_hbm.at[idx2_v.at[pl.ds(pbase, L)]],
                tok2_v.at[pl.ds(pl.multiple_of((rnxt & 1) * L, 8), L)], sem)

        stage_group(pl.multiple_of(wid * ROWS, _RG))
        pre_row(jnp.int32(0))

        def row_body(r, carry0):
            par = r & 1
            pltpu.make_async_copy(tok_hbm.at[pl.ds(0, L)],
                                  tok2_v.at[pl.ds(0, L)], sem).wait()

            nxt = r + 1

            @pl.when(((nxt & (_RG - 1)) == 0) & (nxt < ROWS))
            def _stage():
                stage_group(pl.multiple_of(wid * ROWS, _RG) + (nxt & ~(_RG - 1)))

            pre_row(nxt)

            pbase = par * PAD
            tbase = par * L

            @plsc.parallel_loop(0, L, step=1, unroll=16)
            def tok_loop(t):
                pvec = lax.broadcast(pbase + t, (16,))
                p2 = plsc.load_gather(p2_v, [pvec])
                wsf = (lax.shift_right_logical(p2, 6) & 1).astype(jnp.float32)
                wofs = (lax.shift_right_logical(p2, 2) & ~jnp.int32(63)) + lane
                tvec = lax.broadcast(tbase + t, (16,))
                po = pl.multiple_of(t * D, 8)
                xs = []
                for j in range(4):
                    tok_j = plsc.load_gather(tok2_v, [tvec, lane + 16 * j])
                    pos_j = pos_v[pl.ds(po + 16 * j, 16)]
                    wrd_j = plsc.load_gather(word_v, [wofs + 16 * j])
                    xs.append((tok_j + pos_j) + (wrd_j + ws0[j]) + wsf * wsd[j])
                s = (xs[0] + xs[1]) + (xs[2] + xs[3])
                sq = (xs[0] * xs[0] + xs[1] * xs[1]) + (xs[2] * xs[2] + xs[3] * xs[3])
                meanv = lax.broadcast(jnp.sum(s), (16,)) * jnp.float32(1.0 / D)
                msqv = lax.broadcast(jnp.sum(sq), (16,)) * jnp.float32(1.0 / D)
                var = msqv - meanv * meanv + jnp.float32(1e-5)
                ivar = plsc.bitcast(var, jnp.int32)
                y = plsc.bitcast(jnp.int32(0x5F3759DF) - lax.shift_right_logical(ivar, 1),
                                 jnp.float32)
                for _ in range(2):
                    y = y * (jnp.float32(1.5) - jnp.float32(0.5) * var * y * y)
                for j in range(4):
                    out_v[pl.ds(po + 16 * j, 16)] = (xs[j] - meanv) * y * gam[j] + bet[j]

            b = wid * ROWS + r
            pltpu.sync_copy(out_v,
                            out_hbm.at[pl.ds(pl.multiple_of(b * L * D, 8), L * D)])
            return carry0

        lax.fori_loop(0, ROWS, row_body, jnp.int32(0))
        pltpu.make_async_copy(tok_hbm.at[pl.ds(0, L)],
                              tok2_v.at[pl.ds(0, L)], sem).wait()

    return body


def kernel(input_ids, word_start, token_emb, pos_emb, word_start_emb,
           word_emb, ln_gamma, ln_beta):
    B, L = input_ids.shape
    V, D = token_emb.shape
    M = word_emb.shape[0]
    LP = ((L + 15) // 16) * 16

    ids = input_ids.astype(jnp.int32)
    ws = word_start.astype(jnp.int32)

    mesh = plsc.VectorSubcoreMesh(core_axis_name="c", subcore_axis_name="s",
                                  num_cores=_NC, num_subcores=_NS)
    scratch = [
        pltpu.VMEM((_RG, L), jnp.int32),
        pltpu.VMEM((_RG, L), jnp.int32),
        pltpu.VMEM((2 * LP,), jnp.int32),
        pltpu.VMEM((2 * LP,), jnp.int32),
        pltpu.VMEM((2 * L, D), jnp.float32),
        pltpu.VMEM((L * D,), jnp.float32),
        pltpu.VMEM((M * D,), jnp.float32),
        pltpu.VMEM((2 * D,), jnp.float32),
        pltpu.VMEM((D,), jnp.float32),
        pltpu.VMEM((D,), jnp.float32),
        pltpu.VMEM((L * D,), jnp.float32),
        pltpu.SemaphoreType.DMA,
    ]
    smalls = jnp.concatenate([
        pos_emb[:L].reshape(-1), word_emb.reshape(-1),
        word_start_emb.reshape(-1), ln_gamma, ln_beta])
    run = pl.kernel(
        _make_body(B, L, D, V, M),
        out_type=jax.ShapeDtypeStruct((B * L * D,), jnp.float32),
        mesh=mesh,
        scratch_types=scratch,
        compiler_params=pltpu.CompilerParams(needs_layout_passes=False,
                                             use_tc_tiling_on_sc=False),
    )
    out = run(ids, ws, token_emb, smalls)
    return out.reshape(B, L, D)


# speedup vs baseline: 1.0884x; 1.0884x over previous
import jax
import jax.numpy as jnp
from jax import lax
from jax.experimental import pallas as pl
from jax.experimental.pallas import tpu as pltpu
from jax.experimental.pallas import tpu_sc as plsc

_NC, _NS = 2, 16
_NW = _NC * _NS

_CHUNK_A = 104
_RG = 8


def _make_body(B, L, D, V, M):
    ROWS = B // _NW
    NFULL = L // 16
    TAIL0 = L - 16
    TAILLO = TAIL0 - 16 * (NFULL - 1)
    CHB = L - _CHUNK_A

    PAD = ((L + 15) // 16) * 16

    def body(ids_hbm, ws_hbm, tok_hbm, sm_hbm, out_hbm,
             ids8_v, ws8_v, idx2_v, p2_v, tok2_v, pos_v,
             word_v, wse_v, gam_v, bet_v, out_v, sem):
        wid = lax.axis_index("s") * _NC + lax.axis_index("c")
        lane = lax.iota(jnp.int32, 16)

        o_pos, o_word, o_wse, o_gam, o_bet = (
            0, L * D, L * D + M * D, L * D + M * D + 2 * D,
            L * D + M * D + 2 * D + D)
        pltpu.sync_copy(sm_hbm.at[pl.ds(o_pos, L * D)], pos_v)
        pltpu.sync_copy(sm_hbm.at[pl.ds(o_word, M * D)], word_v)
        pltpu.sync_copy(sm_hbm.at[pl.ds(o_wse, 2 * D)], wse_v)
        pltpu.sync_copy(sm_hbm.at[pl.ds(o_gam, D)], gam_v)
        pltpu.sync_copy(sm_hbm.at[pl.ds(o_bet, D)], bet_v)

        ws0 = [wse_v[pl.ds(16 * j, 16)] for j in range(4)]
        wsd = [wse_v[pl.ds(D + 16 * j, 16)] - ws0[j] for j in range(4)]
        gam = [gam_v[pl.ds(16 * j, 16)] for j in range(4)]
        bet = [bet_v[pl.ds(16 * j, 16)] for j in range(4)]

        def stage_group(b8):
            b8 = pl.multiple_of(b8, _RG)
            pltpu.sync_copy(ids_hbm.at[pl.ds(b8, _RG)], ids8_v)
            pltpu.sync_copy(ws_hbm.at[pl.ds(b8, _RG)], ws8_v)

        def pre_row(rnxt):
            i = rnxt & (_RG - 1)
            pbase = pl.multiple_of((rnxt & 1) * PAD, 16)

            def do_chunk(dst16, v, idv, runbc):
                cum = plsc.cumsum(v) + runbc
                p2_v[dst16] = (lax.shift_left(cum, 8)
                               | lax.shift_left(idv & 1, 7)
                               | lax.shift_left(v, 6))
                idx2_v[dst16] = lax.shift_right_logical(idv, 1)

            run = jnp.int32(0)
            s_tail = jnp.int32(0)
            for k in range(NFULL):
                v = ws8_v[i, pl.ds(16 * k, 16)]
                idv = ids8_v[i, pl.ds(16 * k, 16)]
                do_chunk(pl.ds(pbase + 16 * k, 16), v, idv, lax.broadcast(run, (16,)))
                if k == NFULL - 1:
                    s_tail = run + jnp.sum(jnp.where(lane < TAILLO, v, 0))
                run = run + jnp.sum(v)
            if L > 16 * NFULL:
                vB = ws8_v[i, pl.ds(TAIL0, 16)]
                idvB = ids8_v[i, pl.ds(TAIL0, 16)]
                do_chunk(pl.ds(pbase + TAIL0, 16), vB, idvB,
                         lax.broadcast(s_tail, (16,)))

            pltpu.async_copy(
                tok_hbm.at[idx2_v.at[pl.ds(pbase, L)]],
                tok2_v.at[pl.ds(pl.multiple_of((rnxt & 1) * L, 8), L)], sem)

        stage_group(pl.multiple_of(wid * ROWS, _RG))
        pre_row(jnp.int32(0))

        def row_body(r, carry0):
            par = r & 1
            pltpu.make_async_copy(tok_hbm.at[pl.ds(0, L)],
                                  tok2_v.at[pl.ds(0, L)], sem).wait()

            nxt = r + 1

            @pl.when(((nxt & (_RG - 1)) == 0) & (nxt < ROWS))
            def _stage():
                stage_group(pl.multiple_of(wid * ROWS, _RG) + (nxt & ~(_RG - 1)))

            pre_row(nxt)

            pbase = par * PAD
            tbase = par * L

            @plsc.parallel_loop(0, L, step=1, unroll=16)
            def tok_loop(t):
                pvec = lax.broadcast(pbase + t, (16,))
                p2 = plsc.load_gather(p2_v, [pvec])
                wsf = (lax.shift_right_logical(p2, 6) & 1).astype(jnp.float32)
                colb = lax.shift_right_logical(p2, 1) & 64
                wofs = (lax.shift_right_logical(p2, 2) & ~jnp.int32(63)) + lane
                tvec = lax.broadcast(tbase + t, (16,))
                po = pl.multiple_of(t * D, 8)
                xs = []
                for j in range(4):
                    tok_j = plsc.load_gather(tok2_v, [tvec, colb + (lane + 16 * j)])
                    pos_j = pos_v[pl.ds(po + 16 * j, 16)]
                    wrd_j = plsc.load_gather(word_v, [wofs + 16 * j])
                    xs.append((tok_j + pos_j) + (wrd_j + ws0[j]) + wsf * wsd[j])
                s = (xs[0] + xs[1]) + (xs[2] + xs[3])
                sq = (xs[0] * xs[0] + xs[1] * xs[1]) + (xs[2] * xs[2] + xs[3] * xs[3])
                meanv = lax.broadcast(jnp.sum(s), (16,)) * jnp.float32(1.0 / D)
                msqv = lax.broadcast(jnp.sum(sq), (16,)) * jnp.float32(1.0 / D)
                var = msqv - meanv * meanv + jnp.float32(1e-5)
                ivar = plsc.bitcast(var, jnp.int32)
                y = plsc.bitcast(jnp.int32(0x5F3759DF) - lax.shift_right_logical(ivar, 1),
                                 jnp.float32)
                for _ in range(2):
                    y = y * (jnp.float32(1.5) - jnp.float32(0.5) * var * y * y)
                for j in range(4):
                    out_v[t, pl.ds(16 * j, 16)] = (xs[j] - meanv) * y * gam[j] + bet[j]

            b = wid * ROWS + r
            pltpu.sync_copy(out_v, out_hbm.at[b])
            return carry0

        lax.fori_loop(0, ROWS, row_body, jnp.int32(0))
        pltpu.make_async_copy(tok_hbm.at[pl.ds(0, L)],
                              tok2_v.at[pl.ds(0, L)], sem).wait()

    return body


def kernel(input_ids, word_start, token_emb, pos_emb, word_start_emb,
           word_emb, ln_gamma, ln_beta):
    B, L = input_ids.shape
    V, D = token_emb.shape
    M = word_emb.shape[0]
    LP = ((L + 15) // 16) * 16

    ids = input_ids.astype(jnp.int32)
    ws = word_start.astype(jnp.int32)

    mesh = plsc.VectorSubcoreMesh(core_axis_name="c", subcore_axis_name="s",
                                  num_cores=_NC, num_subcores=_NS)
    scratch = [
        pltpu.VMEM((_RG, L), jnp.int32),
        pltpu.VMEM((_RG, L), jnp.int32),
        pltpu.VMEM((2 * LP,), jnp.int32),
        pltpu.VMEM((2 * LP,), jnp.int32),
        pltpu.VMEM((2 * L, 2 * D), jnp.float32),
        pltpu.VMEM((L * D,), jnp.float32),
        pltpu.VMEM((M * D,), jnp.float32),
        pltpu.VMEM((2 * D,), jnp.float32),
        pltpu.VMEM((D,), jnp.float32),
        pltpu.VMEM((D,), jnp.float32),
        pltpu.VMEM((L, D), jnp.float32),
        pltpu.SemaphoreType.DMA,
    ]
    smalls = jnp.concatenate([
        pos_emb[:L].reshape(-1), word_emb.reshape(-1),
        word_start_emb.reshape(-1), ln_gamma, ln_beta])
    run = pl.kernel(
        _make_body(B, L, D, V, M),
        out_type=jax.ShapeDtypeStruct((B, L, D), jnp.float32),
        mesh=mesh,
        scratch_types=scratch,
        compiler_params=pltpu.CompilerParams(needs_layout_passes=False),
    )
    return run(ids, ws, token_emb.reshape(V // 2, 2 * D), smalls)


# speedup vs baseline: 1.0886x; 1.0002x over previous
import jax
import jax.numpy as jnp
from jax import lax
from jax.experimental import pallas as pl
from jax.experimental.pallas import tpu as pltpu
from jax.experimental.pallas import tpu_sc as plsc

_NC, _NS = 2, 16
_NW = _NC * _NS

_RG = 8


def _make_body(B, L, D, V, M):
    ROWS = B // _NW
    NFULL = L // 16
    TAIL0 = L - 16
    TAILLO = TAIL0 - 16 * (NFULL - 1)
    PAD = ((L + 15) // 16) * 16

    def body(ids_hbm, ws_hbm, tok_hbm, sm_hbm, out_hbm,
             ids8_v, ws8_v, idx2_v, p2_v, tok2_v, pos_v,
             word_v, wse_v, gam_v, bet_v, out_v, sem):
        wid = lax.axis_index("s") * _NC + lax.axis_index("c")
        lane = lax.iota(jnp.int32, 16)

        o_pos, o_word, o_wse, o_gam, o_bet = (
            0, L * D, L * D + M * D, L * D + M * D + 2 * D,
            L * D + M * D + 2 * D + D)
        pltpu.sync_copy(sm_hbm.at[pl.ds(o_pos, L * D)], pos_v)
        pltpu.sync_copy(sm_hbm.at[pl.ds(o_word, M * D)], word_v)
        pltpu.sync_copy(sm_hbm.at[pl.ds(o_wse, 2 * D)], wse_v)
        pltpu.sync_copy(sm_hbm.at[pl.ds(o_gam, D)], gam_v)
        pltpu.sync_copy(sm_hbm.at[pl.ds(o_bet, D)], bet_v)

        ws0 = [wse_v[pl.ds(16 * j, 16)] for j in range(4)]
        wsd = [wse_v[pl.ds(D + 16 * j, 16)] - ws0[j] for j in range(4)]
        gam = [gam_v[pl.ds(16 * j, 16)] for j in range(4)]
        bet = [bet_v[pl.ds(16 * j, 16)] for j in range(4)]

        def stage_group(b8):
            b8 = pl.multiple_of(b8, _RG)
            pltpu.sync_copy(ids_hbm.at[pl.ds(b8, _RG)], ids8_v)
            pltpu.sync_copy(ws_hbm.at[pl.ds(b8, _RG)], ws8_v)

        def pre_row(rnxt):
            i = rnxt & (_RG - 1)
            pbase = pl.multiple_of((rnxt & 1) * PAD, 16)

            def do_chunk(dst16, v, idv, runbc):
                cum = plsc.cumsum(v) + runbc
                p2_v[dst16] = (lax.shift_left(cum, 8)
                               | lax.shift_left(idv & 1, 7)
                               | lax.shift_left(v, 6))
                idx2_v[dst16] = lax.shift_right_logical(idv, 1)

            run = jnp.int32(0)
            s_tail = jnp.int32(0)
            for k in range(NFULL):
                v = ws8_v[i, pl.ds(16 * k, 16)]
                idv = ids8_v[i, pl.ds(16 * k, 16)]
                do_chunk(pl.ds(pbase + 16 * k, 16), v, idv, lax.broadcast(run, (16,)))
                if k == NFULL - 1:
                    s_tail = run + jnp.sum(jnp.where(lane < TAILLO, v, 0))
                run = run + jnp.sum(v)
            if L > 16 * NFULL:
                vB = ws8_v[i, pl.ds(TAIL0, 16)]
                idvB = ids8_v[i, pl.ds(TAIL0, 16)]
                do_chunk(pl.ds(pbase + TAIL0, 16), vB, idvB,
                         lax.broadcast(s_tail, (16,)))

            pltpu.async_copy(
                tok_hbm.at[idx2_v.at[pl.ds(pbase, L)]],
                tok2_v.at[pl.ds(pl.multiple_of((rnxt & 1) * L, 8), L)], sem)

        stage_group(pl.multiple_of(wid * ROWS, _RG))
        pre_row(jnp.int32(0))

        def row_body(r, carry0):
            par = r & 1
            pltpu.make_async_copy(tok_hbm.at[pl.ds(0, L)],
                                  tok2_v.at[pl.ds(0, L)], sem).wait()

            nxt = r + 1

            @pl.when(((nxt & (_RG - 1)) == 0) & (nxt < ROWS))
            def _stage():
                stage_group(pl.multiple_of(wid * ROWS, _RG) + (nxt & ~(_RG - 1)))

            pre_row(nxt)

            pbase = par * PAD
            tbase = par * L

            @plsc.parallel_loop(0, L, step=1, unroll=16)
            def tok_loop(t):
                pvec = lax.broadcast(pbase + t, (16,))
                p2 = plsc.load_gather(p2_v, [pvec])
                wsf = (lax.shift_right_logical(p2, 6) & 1).astype(jnp.float32)
                colb = lax.shift_right_logical(p2, 1) & 64
                wofs = (lax.shift_right_logical(p2, 2) & ~jnp.int32(63)) + lane
                tvec = lax.broadcast(tbase + t, (16,))
                po = pl.multiple_of(t * D, 8)
                xs = []
                for j in range(4):
                    tok_j = plsc.load_gather(tok2_v, [tvec, colb + (lane + 16 * j)])
                    pos_j = pos_v[pl.ds(po + 16 * j, 16)]
                    wrd_j = plsc.load_gather(word_v, [wofs + 16 * j])
                    xs.append((tok_j + pos_j) + (wrd_j + ws0[j]) + wsf * wsd[j])
                s = (xs[0] + xs[1]) + (xs[2] + xs[3])
                sq = (xs[0] * xs[0] + xs[1] * xs[1]) + (xs[2] * xs[2] + xs[3] * xs[3])
                meanv = lax.broadcast(jnp.sum(s), (16,)) * jnp.float32(1.0 / D)
                msqv = lax.broadcast(jnp.sum(sq), (16,)) * jnp.float32(1.0 / D)
                var = msqv - meanv * meanv + jnp.float32(1e-5)
                ivar = plsc.bitcast(var, jnp.int32)
                y = plsc.bitcast(jnp.int32(0x5F3759DF) - lax.shift_right_logical(ivar, 1),
                                 jnp.float32)
                for _ in range(2):
                    y = y * (jnp.float32(1.5) - jnp.float32(0.5) * var * y * y)
                for j in range(4):
                    out_v[t, pl.ds(16 * j, 16)] = (xs[j] - meanv) * y * gam[j] + bet[j]

            b = wid * ROWS + r
            pltpu.sync_copy(out_v, out_hbm.at[b])
            return carry0

        lax.fori_loop(0, ROWS, row_body, jnp.int32(0))
        pltpu.make_async_copy(tok_hbm.at[pl.ds(0, L)],
                              tok2_v.at[pl.ds(0, L)], sem).wait()

    return body


def kernel(input_ids, word_start, token_emb, pos_emb, word_start_emb,
           word_emb, ln_gamma, ln_beta):
    B, L = input_ids.shape
    V, D = token_emb.shape
    M = word_emb.shape[0]
    LP = ((L + 15) // 16) * 16

    ids = input_ids.astype(jnp.int32)
    ws = word_start.astype(jnp.int32)

    mesh = plsc.VectorSubcoreMesh(core_axis_name="c", subcore_axis_name="s",
                                  num_cores=_NC, num_subcores=_NS)
    scratch = [
        pltpu.VMEM((_RG, L), jnp.int32),
        pltpu.VMEM((_RG, L), jnp.int32),
        pltpu.VMEM((2 * LP,), jnp.int32),
        pltpu.VMEM((2 * LP,), jnp.int32),
        pltpu.VMEM((2 * L, 2 * D), jnp.float32),
        pltpu.VMEM((L * D,), jnp.float32),
        pltpu.VMEM((M * D,), jnp.float32),
        pltpu.VMEM((2 * D,), jnp.float32),
        pltpu.VMEM((D,), jnp.float32),
        pltpu.VMEM((D,), jnp.float32),
        pltpu.VMEM((L, D), jnp.float32),
        pltpu.SemaphoreType.DMA,
    ]
    smalls = jnp.concatenate([
        pos_emb[:L].reshape(-1), word_emb.reshape(-1),
        word_start_emb.reshape(-1), ln_gamma, ln_beta])
    run = pl.kernel(
        _make_body(B, L, D, V, M),
        out_type=jax.ShapeDtypeStruct((B, L, D), jnp.float32),
        mesh=mesh,
        scratch_types=scratch,
        compiler_params=pltpu.CompilerParams(needs_layout_passes=False),
    )
    return run(ids, ws, token_emb.reshape(V // 2, 2 * D), smalls)
